# Initial kernel scaffold; baseline (speedup 1.0000x reference)
import jax
import jax.numpy as jnp
from jax.experimental import pallas as pl


def kernel(x, importance, W_all, b_all, neuron_emb):
    raise NotImplementedError("write your pallas kernel here")


# speedup vs baseline: 5.2501x; 5.2501x over previous
import jax
import jax.numpy as jnp
from jax.experimental import pallas as pl
from jax.experimental.pallas import tpu as pltpu

_B, _S = 1, 2048
_D_MODEL = 1024
_D_SPACE = 64
_N_POOL = 1024
_N_GROUPS = 6
_TOP_K = 8


def _router_body(x_ref, imp_ref, w_ref, b_ref, emb_ref, out_ref):
    x = x_ref[...]
    imp = imp_ref[...]
    impn = imp / (jnp.sum(imp) + 1e-8)
    ii = jax.lax.broadcasted_iota(jnp.int32, (1, _N_POOL), 1)

    for i in range(_N_GROUPS):
        w = w_ref[:, i * _D_SPACE:(i + 1) * _D_SPACE]
        b = b_ref[:, i * _D_SPACE:(i + 1) * _D_SPACE]
        emb = emb_ref[i * _N_POOL:(i + 1) * _N_POOL, :]

        nrm = jnp.sqrt(jnp.sum(emb * emb, axis=1, keepdims=True))
        emb_n = emb / jnp.maximum(nrm, 1e-12)

        h = jnp.dot(x, w, preferred_element_type=jnp.float32) + b
        logits = jax.lax.dot_general(
            h, emb_n, (((1,), (1,)), ((), ())),
            preferred_element_type=jnp.float32)

        m = jnp.max(logits, axis=1, keepdims=True)
        e = jnp.exp(logits - m)
        z = jnp.sum(e, axis=1, keepdims=True)
        pooled = jnp.sum(e * (impn / z), axis=0, keepdims=True)

        masked = pooled
        sparse = jnp.zeros_like(pooled)
        for _ in range(_TOP_K):
            mval = jnp.max(masked)
            idx = jnp.min(jnp.where(masked == mval, ii, _N_POOL))
            sel = ii == idx
            sparse = jnp.where(sel, pooled, sparse)
            masked = jnp.where(sel, -1.0, masked)
        out = sparse / (jnp.sum(sparse) + 1e-8)
        out_ref[i, :, :] = out


def kernel(x, importance, W_all, b_all, neuron_emb):
    x2 = x.reshape(_S, _D_MODEL)
    imp = importance.reshape(_S, 1)
    b2 = b_all.reshape(1, _N_GROUPS * _D_SPACE)
    emb = neuron_emb[: _N_GROUPS * _N_POOL, :]

    out = pl.pallas_call(
        _router_body,
        out_shape=jax.ShapeDtypeStruct((_N_GROUPS, _B, _N_POOL), jnp.float32),
    )(x2, imp, W_all, b2, emb)
    return out
